# baseline (speedup 1.0000x reference)
import jax
import jax.numpy as jnp
from jax import lax
from jax.experimental import pallas as pl
from jax.experimental.pallas import tpu as pltpu
from jax.experimental.pallas import tpu_sc as plsc

B, C, H, W = 32, 768, 32, 32
P = H * W
K = 8
LANES = 16
CHUNK = 32
GROUPS = CHUNK // LANES
NCHUNKS = P // CHUNK
CBATCH = C // K
NBUF = 4
NC, NS = 2, 16

_SORT8 = [[(0, 1), (2, 3), (4, 5), (6, 7)],
          [(0, 2), (1, 3), (4, 6), (5, 7)],
          [(1, 2), (5, 6)],
          [(0, 4), (1, 5), (2, 6), (3, 7)],
          [(2, 4), (3, 5)],
          [(1, 2), (3, 4), (5, 6)]]
_BITONIC8 = [[(0, 4), (1, 5), (2, 6), (3, 7)],
             [(0, 2), (1, 3), (4, 6), (5, 7)],
             [(0, 1), (2, 3), (4, 5), (6, 7)]]


def _apply_net(vals, net):
    for layer in net:
        for a, b in layer:
            hi = jnp.maximum(vals[a], vals[b])
            lo = jnp.minimum(vals[a], vals[b])
            vals[a], vals[b] = hi, lo
    return vals


def _sc_body(x_hbm, out_hbm, bufs, in_sems, out_sems):
    wid = lax.axis_index("s") * NC + lax.axis_index("c")

    def issue_in(q, b):
        pltpu.async_copy(x_hbm.at[wid, :, pl.ds(q * CHUNK, CHUNK)],
                         bufs.at[b], in_sems.at[b])

    def wait_in(q, b):
        pltpu.make_async_copy(x_hbm.at[wid, :, pl.ds(q * CHUNK, CHUNK)],
                              bufs.at[b], in_sems.at[b]).wait()

    def issue_out(q, b):
        pltpu.async_copy(bufs.at[b],
                         out_hbm.at[wid, :, pl.ds(q * CHUNK, CHUNK)],
                         out_sems.at[b])

    def wait_out(q, b):
        pltpu.make_async_copy(bufs.at[b],
                              out_hbm.at[wid, :, pl.ds(q * CHUNK, CHUNK)],
                              out_sems.at[b]).wait()

    issue_in(0, 0)
    issue_in(1, 1)

    @pl.loop(0, NCHUNKS, step=NBUF)
    def _grp(j):
        for b in range(NBUF):
            q = j + b
            wait_in(q, b)
            buf = bufs.at[b]

            neg = jnp.full((LANES,), -jnp.inf, jnp.float32)

            def batch_body(c8, ms):
                ms = list(ms)
                base = c8 * K
                for g in range(GROUPS):
                    t = [buf[base + k, g * LANES:(g + 1) * LANES]
                         for k in range(K)]
                    t = _apply_net(t, _SORT8)
                    m = ms[g * K:(g + 1) * K]
                    u = [jnp.maximum(m[i], t[K - 1 - i]) for i in range(K)]
                    u = _apply_net(u, _BITONIC8)
                    ms[g * K:(g + 1) * K] = u
                return tuple(ms)

            ms = plsc.parallel_loop(
                0, CBATCH,
                carry=tuple(neg for _ in range(GROUPS * K)))(batch_body)
            thr = [ms[g * K + K - 1] for g in range(GROUPS)]

            zero = jnp.zeros((LANES,), jnp.float32)

            @plsc.parallel_loop(0, C, unroll=2)
            def mask_body(c):
                for g in range(GROUPS):
                    t = buf[c, g * LANES:(g + 1) * LANES]
                    buf[c, g * LANES:(g + 1) * LANES] = jnp.where(
                        t >= thr[g], t, zero)

            issue_out(q, b)

            nq = q + 2
            nb = (b + 2) % NBUF

            @pl.when(nq < NCHUNKS)
            def _refill():
                @pl.when(q >= 2)
                def _drain():
                    wait_out(q - 2, nb)
                issue_in(nq, nb)

    for b in range(NBUF):
        q = NCHUNKS - NBUF + b
        wait_out(q, b % NBUF)


@jax.jit
def kernel(x):
    xr = x.reshape(B, C, P)
    run = pl.kernel(
        _sc_body,
        out_type=jax.ShapeDtypeStruct((B, C, P), jnp.float32),
        mesh=plsc.VectorSubcoreMesh(core_axis_name="c", subcore_axis_name="s"),
        scratch_types=[
            pltpu.VMEM((NBUF, C, CHUNK), jnp.float32),
            pltpu.SemaphoreType.DMA((NBUF,)),
            pltpu.SemaphoreType.DMA((NBUF,)),
        ],
        compiler_params=pltpu.CompilerParams(use_tc_tiling_on_sc=False),
    )
    return run(xr).reshape(B, C, H, W)
